# baseline (speedup 1.0000x reference)
import jax
import jax.numpy as jnp
from jax import lax
from jax.experimental import pallas as pl
from jax.experimental.pallas import tpu as pltpu

_NO_PATCHES = 14
_P = _NO_PATCHES * _NO_PATCHES
_PATCH = 16
_HW = _NO_PATCHES * _PATCH


def _fused_kernel(pct_ref, s_ref, x_ref, o_ref, mfull_ref):
    j = pl.program_id(1)

    @pl.when(j == 0)
    def _():
        pct = pct_ref[0, 0]
        p_eff = jnp.where(pct == 0.0, jnp.float32(0.0),
                          jnp.maximum(pct, jnp.float32(0.07)))
        num = jnp.floor(p_eff * jnp.float32(_P))

        s = s_ref[0, 0]
        s_p = s[:, None]
        s_q = s[None, :]
        q_idx = lax.broadcasted_iota(jnp.int32, (_P, _P), 1)
        p_idx = lax.broadcasted_iota(jnp.int32, (_P, _P), 0)
        beats = (s_q > s_p) | ((s_q == s_p) & (q_idx < p_idx))
        rank = jnp.sum(beats.astype(jnp.float32), axis=1)
        mask_bp = (rank >= num).astype(jnp.float32)

        ii = lax.broadcasted_iota(jnp.int32, (_HW, _P), 0) // _PATCH
        pp_v = lax.broadcasted_iota(jnp.int32, (_HW, _P), 1) // _NO_PATCHES
        V = (pp_v == ii).astype(jnp.float32)
        pi = lax.broadcasted_iota(jnp.int32, (_P, _HW), 0)
        pm = pi - _NO_PATCHES * (pi // _NO_PATCHES)
        jj = lax.broadcasted_iota(jnp.int32, (_P, _HW), 1) // _PATCH
        U = (pm == jj).astype(jnp.float32)
        scaled = V * mask_bp[None, :]
        mfull_ref[...] = jnp.dot(scaled, U, preferred_element_type=jnp.float32)

    o_ref[...] = x_ref[...] * mfull_ref[...][None, None]


def kernel(x, percentage, probabilities):
    b, c, H, W = x.shape
    key = jax.random.key(42)
    u = jax.random.uniform(key, probabilities.shape, minval=1e-20, maxval=1.0)
    gumbel = -jnp.log(-jnp.log(u))
    scores = jnp.log(probabilities) + gumbel
    scores3 = scores.reshape(b, 1, _P)
    pct = jnp.reshape(percentage.astype(jnp.float32), (1, 1))

    CC = 24
    out = pl.pallas_call(
        _fused_kernel,
        out_shape=jax.ShapeDtypeStruct(x.shape, x.dtype),
        grid=(b, c // CC),
        in_specs=[
            pl.BlockSpec(memory_space=pltpu.MemorySpace.SMEM),
            pl.BlockSpec((1, 1, _P), lambda i, j: (i, 0, 0)),
            pl.BlockSpec((1, CC, H, W), lambda i, j: (i, j, 0, 0)),
        ],
        out_specs=pl.BlockSpec((1, CC, H, W), lambda i, j: (i, j, 0, 0)),
        scratch_shapes=[pltpu.VMEM((_HW, _HW), jnp.float32)],
    )(pct, scores3, x)
    return out


# speedup vs baseline: 1.0189x; 1.0189x over previous
import jax
import jax.numpy as jnp
from jax import lax
from jax.experimental import pallas as pl
from jax.experimental.pallas import tpu as pltpu

_NO_PATCHES = 14
_P = _NO_PATCHES * _NO_PATCHES
_PATCH = 16
_HW = _NO_PATCHES * _PATCH


def _fused_kernel(pct_ref, s_ref, x_ref, o_ref, mfull_ref):
    j = pl.program_id(1)

    @pl.when(j == 0)
    def _():
        pct = pct_ref[0, 0]
        p_eff = jnp.where(pct == 0.0, jnp.float32(0.0),
                          jnp.maximum(pct, jnp.float32(0.07)))
        num = jnp.floor(p_eff * jnp.float32(_P))

        s = s_ref[0, 0]
        s_p = s[:, None]
        s_q = s[None, :]
        q_idx = lax.broadcasted_iota(jnp.int32, (_P, _P), 1)
        p_idx = lax.broadcasted_iota(jnp.int32, (_P, _P), 0)
        beats = (s_q > s_p) | ((s_q == s_p) & (q_idx < p_idx))
        rank = jnp.sum(beats.astype(jnp.float32), axis=1)
        mask_bp = (rank >= num).astype(jnp.float32)

        ii = lax.broadcasted_iota(jnp.int32, (_HW, _P), 0) // _PATCH
        pp_v = lax.broadcasted_iota(jnp.int32, (_HW, _P), 1) // _NO_PATCHES
        V = (pp_v == ii).astype(jnp.float32)
        pi = lax.broadcasted_iota(jnp.int32, (_P, _HW), 0)
        pm = pi - _NO_PATCHES * (pi // _NO_PATCHES)
        jj = lax.broadcasted_iota(jnp.int32, (_P, _HW), 1) // _PATCH
        U = (pm == jj).astype(jnp.float32)
        scaled = V * mask_bp[None, :]
        mfull_ref[...] = jnp.dot(scaled, U, preferred_element_type=jnp.float32)

    o_ref[...] = x_ref[...] * mfull_ref[...][None, None]


def kernel(x, percentage, probabilities):
    b, c, H, W = x.shape
    key = jax.random.key(42)
    u = jax.random.uniform(key, probabilities.shape, minval=1e-20, maxval=1.0)
    gumbel = -jnp.log(-jnp.log(u))
    scores = jnp.log(probabilities) + gumbel
    scores3 = scores.reshape(b, 1, _P)
    pct = jnp.reshape(percentage.astype(jnp.float32), (1, 1))

    CC = 48
    out = pl.pallas_call(
        _fused_kernel,
        out_shape=jax.ShapeDtypeStruct(x.shape, x.dtype),
        grid=(b, c // CC),
        in_specs=[
            pl.BlockSpec(memory_space=pltpu.MemorySpace.SMEM),
            pl.BlockSpec((1, 1, _P), lambda i, j: (i, 0, 0)),
            pl.BlockSpec((1, CC, H, W), lambda i, j: (i, j, 0, 0)),
        ],
        out_specs=pl.BlockSpec((1, CC, H, W), lambda i, j: (i, j, 0, 0)),
        scratch_shapes=[pltpu.VMEM((_HW, _HW), jnp.float32)],
    )(pct, scores3, x)
    return out
